# baseline (device time: 23271 ns/iter reference)
import functools

import jax
import jax.numpy as jnp
from jax import lax
from jax.experimental import pallas as pl
from jax.experimental.pallas import tpu as pltpu

N_DEV = 8
N_ROUNDS = 3


def kernel(table, idx):
    rows_per, d = table.shape
    n_idx = idx.shape[0]
    idx2 = idx.reshape(n_idx, 1)

    def body(table_ref, idx_ref, out_ref, acc_ref, comm_ref, send_sems, recv_sems):
        my = lax.axis_index("i")

        barrier_sem = pltpu.get_barrier_semaphore()
        for k in range(N_ROUNDS):
            partner = my ^ (1 << k)
            pl.semaphore_signal(
                barrier_sem,
                inc=1,
                device_id=(partner,),
                device_id_type=pl.DeviceIdType.MESH,
            )
        pl.semaphore_wait(barrier_sem, N_ROUNDS)

        ids = idx_ref[:, :]
        col = lax.broadcasted_iota(jnp.int32, (n_idx, rows_per), 1) + my * rows_per
        onehot = (col == ids).astype(jnp.bfloat16)
        tbl = table_ref[:, :].astype(jnp.bfloat16)
        partial = lax.dot_general(
            onehot,
            tbl,
            (((1,), (0,)), ((), ())),
            preferred_element_type=jnp.float32,
        )
        acc_ref[:, :] = partial.astype(jnp.bfloat16)

        for k in range(N_ROUNDS):
            partner = my ^ (1 << k)
            rdma = pltpu.make_async_remote_copy(
                src_ref=acc_ref,
                dst_ref=comm_ref.at[k],
                send_sem=send_sems.at[k],
                recv_sem=recv_sems.at[k],
                device_id=(partner,),
                device_id_type=pl.DeviceIdType.MESH,
            )
            rdma.start()
            rdma.wait()
            acc_ref[:, :] = acc_ref[:, :] + comm_ref[k, :, :]

        out_ref[:, :] = acc_ref[:, :].astype(jnp.float32)

        @functools.partial(pl.run_scoped, exit_sem=pltpu.SemaphoreType.REGULAR)
        def _(exit_sem):
            for k in range(N_ROUNDS):
                partner = my ^ (1 << k)
                pl.semaphore_signal(
                    exit_sem,
                    inc=1,
                    device_id=(partner,),
                    device_id_type=pl.DeviceIdType.MESH,
                )
            pl.semaphore_wait(exit_sem, N_ROUNDS)

    return pl.pallas_call(
        body,
        out_shape=jax.ShapeDtypeStruct((n_idx, d), jnp.float32),
        in_specs=[
            pl.BlockSpec(memory_space=pltpu.VMEM),
            pl.BlockSpec(memory_space=pltpu.VMEM),
        ],
        out_specs=pl.BlockSpec(memory_space=pltpu.VMEM),
        scratch_shapes=[
            pltpu.VMEM((n_idx, d), jnp.bfloat16),
            pltpu.VMEM((N_ROUNDS, n_idx, d), jnp.bfloat16),
            pltpu.SemaphoreType.DMA((N_ROUNDS,)),
            pltpu.SemaphoreType.DMA((N_ROUNDS,)),
        ],
        compiler_params=pltpu.CompilerParams(collective_id=0),
    )(table, idx2)


# device time: 16615 ns/iter; 1.4006x vs baseline; 1.4006x over previous
import jax
import jax.numpy as jnp
from jax import lax
from jax.experimental import pallas as pl
from jax.experimental.pallas import tpu as pltpu

N_DEV = 8


def kernel(table, idx):
    rows_per, d = table.shape
    n_idx = idx.shape[0]
    chunk = n_idx // N_DEV
    idx2 = idx.reshape(n_idx, 1)

    def body(
        table_ref,
        idx_ref,
        out_ref,
        partial_ref,
        rs_ref,
        ag_ref,
        rs_send_sems,
        rs_recv_sems,
        ag_send_sems,
        ag_recv_sems,
    ):
        my = lax.axis_index("i")

        ids = idx_ref[:, :]
        col = lax.broadcasted_iota(jnp.int32, (n_idx, rows_per), 1) + my * rows_per
        onehot = (col == ids).astype(jnp.bfloat16)
        tbl = table_ref[:, :].astype(jnp.bfloat16)
        partial = lax.dot_general(
            onehot,
            tbl,
            (((1,), (0,)), ((), ())),
            preferred_element_type=jnp.float32,
        )
        partial_ref[:, :] = partial.astype(jnp.bfloat16)
        rs_ref[0, :, :] = partial_ref[pl.ds(my * chunk, chunk), :]

        barrier_sem = pltpu.get_barrier_semaphore()
        for t in range(1, N_DEV):
            pl.semaphore_signal(
                barrier_sem,
                inc=1,
                device_id=((my + t) % N_DEV,),
                device_id_type=pl.DeviceIdType.MESH,
            )
        pl.semaphore_wait(barrier_sem, N_DEV - 1)

        rs_sends = []
        for t in range(1, N_DEV):
            p = (my + t) % N_DEV
            rdma = pltpu.make_async_remote_copy(
                src_ref=partial_ref.at[pl.ds(p * chunk, chunk), :],
                dst_ref=rs_ref.at[N_DEV - t],
                send_sem=rs_send_sems.at[t],
                recv_sem=rs_recv_sems.at[N_DEV - t],
                device_id=(p,),
                device_id_type=pl.DeviceIdType.MESH,
            )
            rdma.start()
            rs_sends.append(rdma)

        for s in range(1, N_DEV):
            recv = pltpu.make_async_remote_copy(
                src_ref=rs_ref.at[s],
                dst_ref=rs_ref.at[s],
                send_sem=rs_send_sems.at[s],
                recv_sem=rs_recv_sems.at[s],
                device_id=(my,),
                device_id_type=pl.DeviceIdType.MESH,
            )
            recv.wait_recv()
        summed = jnp.sum(rs_ref[:, :, :].astype(jnp.float32), axis=0)
        ag_ref[pl.ds(my * chunk, chunk), :] = summed.astype(jnp.bfloat16)

        ag_sends = []
        for t in range(1, N_DEV):
            p = (my + t) % N_DEV
            rdma = pltpu.make_async_remote_copy(
                src_ref=ag_ref.at[pl.ds(my * chunk, chunk), :],
                dst_ref=ag_ref.at[pl.ds(my * chunk, chunk), :],
                send_sem=ag_send_sems.at[t],
                recv_sem=ag_recv_sems.at[N_DEV - t],
                device_id=(p,),
                device_id_type=pl.DeviceIdType.MESH,
            )
            rdma.start()
            ag_sends.append(rdma)

        for s in range(1, N_DEV):
            q = (my + s) % N_DEV
            recv = pltpu.make_async_remote_copy(
                src_ref=ag_ref.at[pl.ds(q * chunk, chunk), :],
                dst_ref=ag_ref.at[pl.ds(q * chunk, chunk), :],
                send_sem=ag_send_sems.at[s],
                recv_sem=ag_recv_sems.at[s],
                device_id=(my,),
                device_id_type=pl.DeviceIdType.MESH,
            )
            recv.wait_recv()

        out_ref[:, :] = ag_ref[:, :].astype(jnp.float32)

        for rdma in rs_sends + ag_sends:
            rdma.wait_send()

    return pl.pallas_call(
        body,
        out_shape=jax.ShapeDtypeStruct((n_idx, d), jnp.float32),
        in_specs=[
            pl.BlockSpec(memory_space=pltpu.VMEM),
            pl.BlockSpec(memory_space=pltpu.VMEM),
        ],
        out_specs=pl.BlockSpec(memory_space=pltpu.VMEM),
        scratch_shapes=[
            pltpu.VMEM((n_idx, d), jnp.bfloat16),
            pltpu.VMEM((N_DEV, chunk, d), jnp.bfloat16),
            pltpu.VMEM((n_idx, d), jnp.bfloat16),
            pltpu.SemaphoreType.DMA((N_DEV,)),
            pltpu.SemaphoreType.DMA((N_DEV,)),
            pltpu.SemaphoreType.DMA((N_DEV,)),
            pltpu.SemaphoreType.DMA((N_DEV,)),
        ],
        compiler_params=pltpu.CompilerParams(collective_id=0),
    )(table, idx2)
